# baseline (speedup 1.0000x reference)
import functools

import jax
import jax.numpy as jnp
from jax import lax
from jax.experimental import pallas as pl
from jax.experimental.pallas import tpu as pltpu
from jax.experimental.pallas import tpu_sc as plsc

B, N, D = 2, 4096, 2048
H, KVH = 16, 4
DH = D // H
HD, NH = 8, 4
BS = 128
NB = (N + BS - 1) // BS
REP = H // KVH

TB = 256
TT = (B * N) // TB
TBO = 512
TTO = (B * N) // TBO


def _silu(t):
    return t * jax.nn.sigmoid(t)


def _full(shape):
    return pl.BlockSpec(shape, lambda i: (0,) * len(shape))


def _encA_kernel(x_ref, xb_ref, winp_ref, cw_ref, cb_ref, wv_ref,
                 zz_ref, v_ref):
    x = x_ref[...]
    xb = xb_ref[0]
    zg = lax.dot_general(x, winp_ref[...], (((1,), (1,)), ((), ())))
    z = zg[:, :D]
    gate = zg[:, D:]
    zb = lax.dot_general(xb, winp_ref[:D], (((1,), (1,)), ((), ())))
    z_up = jnp.concatenate([zb[0:1], z[:-1]], axis=0)
    z_dn = jnp.concatenate([z[1:], zb[1:2]], axis=0)
    zc = (z_up * cw_ref[0:1] + z * cw_ref[1:2] + z_dn * cw_ref[2:3]
          + cb_ref[0:1])
    zz_ref[...] = _silu(zc) * _silu(gate)
    v_ref[...] = lax.dot_general(x, wv_ref[...], (((1,), (1,)), ((), ())))


def _encB_kernel(zz_ref, wout_ref, qkg_ref):
    qkg_ref[...] = lax.dot_general(zz_ref[...], wout_ref[...],
                                   (((1,), (1,)), ((), ())))


def _encC_kernel(qg_ref, kg_ref, wq_ref, wk_ref, q_ref, k_ref):
    q_ref[...] = lax.dot_general(qg_ref[...], wq_ref[...],
                                 (((1,), (1,)), ((), ())))
    k_ref[...] = lax.dot_general(kg_ref[...], wk_ref[...],
                                 (((1,), (1,)), ((), ())))


def _enc_a(x2, xb, winp_t, cw, cb2, wv_t):
    return pl.pallas_call(
        _encA_kernel,
        grid=(TT,),
        in_specs=[
            pl.BlockSpec((TB, D), lambda i: (i, 0)),
            pl.BlockSpec((1, 2, D), lambda i: (i, 0, 0)),
            _full((2 * D, D)),
            _full((3, D)),
            _full((1, D)),
            _full((KVH * DH, D)),
        ],
        out_specs=[
            pl.BlockSpec((TB, D), lambda i: (i, 0)),
            pl.BlockSpec((TB, KVH * DH), lambda i: (i, 0)),
        ],
        out_shape=[
            jax.ShapeDtypeStruct((B * N, D), jnp.float32),
            jax.ShapeDtypeStruct((B * N, KVH * DH), jnp.float32),
        ],
    )(x2, xb, winp_t, cw, cb2, wv_t)


def _enc_bc(zz, wout_t, wq_t, wk_t):
    qkg = pl.pallas_call(
        _encB_kernel,
        grid=(TT,),
        in_specs=[
            pl.BlockSpec((TB, D), lambda i: (i, 0)),
            _full((2 * D, D)),
        ],
        out_specs=pl.BlockSpec((TB, 2 * D), lambda i: (i, 0)),
        out_shape=jax.ShapeDtypeStruct((B * N, 2 * D), jnp.float32),
    )(zz, wout_t)
    q2, k2 = pl.pallas_call(
        _encC_kernel,
        grid=(TT,),
        in_specs=[
            pl.BlockSpec((TB, D), lambda i: (i, 0)),
            pl.BlockSpec((TB, D), lambda i: (i, 1)),
            _full((H * DH, D)),
            _full((KVH * DH, D)),
        ],
        out_specs=[
            pl.BlockSpec((TB, H * DH), lambda i: (i, 0)),
            pl.BlockSpec((TB, KVH * DH), lambda i: (i, 0)),
        ],
        out_shape=[
            jax.ShapeDtypeStruct((B * N, H * DH), jnp.float32),
            jax.ShapeDtypeStruct((B * N, KVH * DH), jnp.float32),
        ],
    )(qkg, qkg, wq_t, wk_t)
    return q2, k2


def _att_kernel(q_ref, k_ref, v_ref, o_ref):
    q = q_ref[0]
    k = k_ref[0]
    v = v_ref[0]
    scale = 1.0 / jnp.sqrt(jnp.float32(DH))
    for g in range(KVH):
        kh = k[:, g * DH:(g + 1) * DH]
        vh = v[:, g * DH:(g + 1) * DH]
        q4 = jnp.concatenate(
            [q[:, (g * REP + j) * DH:(g * REP + j + 1) * DH]
             for j in range(REP)], axis=0)
        s = jax.lax.dot_general(q4, kh, (((1,), (1,)), ((), ()))) * scale
        m = jnp.max(s, axis=1, keepdims=True)
        p = jnp.exp(s - m)
        a = p / jnp.sum(p, axis=1, keepdims=True)
        o4 = jnp.dot(a, vh)
        for j in range(REP):
            h = g * REP + j
            o_ref[0, :, h * DH:(h + 1) * DH] = o4[j * BS:(j + 1) * BS]


def _attention(qs, ks, vs):
    return pl.pallas_call(
        _att_kernel,
        grid=(B * NB,),
        in_specs=[
            pl.BlockSpec((1, BS, H * DH), lambda i: (i, 0, 0)),
            pl.BlockSpec((1, BS, KVH * DH), lambda i: (i, 0, 0)),
            pl.BlockSpec((1, BS, KVH * DH), lambda i: (i, 0, 0)),
        ],
        out_specs=pl.BlockSpec((1, BS, H * DH), lambda i: (i, 0, 0)),
        out_shape=jax.ShapeDtypeStruct((B * NB, BS, H * DH), jnp.float32),
    )(qs, ks, vs)


def _out_kernel(o_ref, wo_ref, rw_ref, y_ref):
    oo = lax.dot_general(o_ref[...], wo_ref[...], (((1,), (1,)), ((), ())))
    rms = jnp.sqrt(jnp.mean(oo * oo, axis=-1, keepdims=True) + 1e-8)
    y_ref[...] = oo / rms * rw_ref[0:1]


def _outproj(o_s, wo_t, rw2):
    return pl.pallas_call(
        _out_kernel,
        grid=(TTO,),
        in_specs=[
            pl.BlockSpec((TBO, D), lambda i: (i, 0)),
            pl.BlockSpec((D, D), lambda i: (0, 0)),
            pl.BlockSpec((1, D), lambda i: (0, 0)),
        ],
        out_specs=pl.BlockSpec((TBO, D), lambda i: (i, 0)),
        out_shape=jax.ShapeDtypeStruct((B * N, D), jnp.float32),
    )(o_s, wo_t, rw2)


_CHUNK = (B * N) // 32
_RCH = 16


def _sc_mesh():
    return plsc.VectorSubcoreMesh(core_axis_name="c", subcore_axis_name="s")


def _sc_sort(ids_flat):
    nv = _CHUNK // 16

    @functools.partial(
        pl.kernel,
        out_type=jax.ShapeDtypeStruct((32 * NB,), jnp.int32),
        mesh=_sc_mesh(),
        compiler_params=pltpu.CompilerParams(needs_layout_passes=False),
        scratch_types=[
            pltpu.VMEM((_CHUNK,), jnp.int32),
            pltpu.VMEM((NB,), jnp.int32),
        ],
    )
    def histk(ids_hbm, hists_hbm, ids_v, hist_v):
        c = lax.axis_index("c")
        s = lax.axis_index("s")
        w = c * 16 + s
        base = c * N + s * _CHUNK
        pltpu.sync_copy(ids_hbm.at[pl.ds(base, _CHUNK)], ids_v)
        lanes = lax.iota(jnp.int32, 16)

        def hbody(b, carry):
            cnt = jnp.int32(0)
            for g in range(nv):
                cnt += jnp.sum((ids_v[pl.ds(g * 16, 16)] == b)
                               .astype(jnp.int32))
            half = b // 16
            lane = b - half * 16
            m_lane = (lanes == lane).astype(jnp.int32)
            for hh in range(2):
                eq = jnp.int32(1) - jnp.minimum(jnp.abs(half - hh), 1)
                cur = hist_v[pl.ds(hh * 16, 16)]
                hist_v[pl.ds(hh * 16, 16)] = cur + m_lane * eq * (cnt - cur)
            return carry
        lax.fori_loop(0, NB, hbody, 0)
        pltpu.sync_copy(hist_v, hists_hbm.at[pl.ds(w * NB, NB)])

    hists = histk(ids_flat)

    @functools.partial(
        pl.kernel,
        out_type=jax.ShapeDtypeStruct((B * N,), jnp.int32),
        mesh=_sc_mesh(),
        compiler_params=pltpu.CompilerParams(needs_layout_passes=False),
        scratch_types=[
            pltpu.VMEM((_CHUNK,), jnp.int32),
            pltpu.VMEM((16 * NB,), jnp.int32),
            pltpu.VMEM((_CHUNK,), jnp.int32),
        ],
    )
    def sortk(ids_hbm, hists_hbm, inv_hbm, ids_v, allh_v, inv_v):
        c = lax.axis_index("c")
        s = lax.axis_index("s")
        base = c * N + s * _CHUNK
        pltpu.sync_copy(ids_hbm.at[pl.ds(base, _CHUNK)], ids_v)
        pltpu.sync_copy(hists_hbm.at[pl.ds(c * 16 * NB, 16 * NB)], allh_v)
        lanes = lax.iota(jnp.int32, 16)

        tot = [jnp.zeros((16,), jnp.int32) for _ in range(2)]
        part = [jnp.zeros((16,), jnp.int32) for _ in range(2)]
        for t in range(16):
            sel = jnp.minimum(jnp.maximum(s - t, 0), 1)
            for hh in range(2):
                row = allh_v[pl.ds(t * NB + hh * 16, 16)]
                tot[hh] = tot[hh] + row
                part[hh] = part[hh] + row * sel
        excl = []
        run = c * N
        for hh in range(2):
            cs = plsc.cumsum(tot[hh])
            excl.append(run + cs - tot[hh] + part[hh])
            run = run + jnp.sum(tot[hh])

        def pbody(b, carry):
            half = b // 16
            lane = b - half * 16
            m_lane = (lanes == lane).astype(jnp.int32)
            offv = excl[0] + (excl[1] - excl[0]) * half
            pos = jnp.sum(offv * m_lane)
            for g in range(nv):
                ids16 = ids_v[pl.ds(g * 16, 16)]
                m16 = (ids16 == b)
                mi16 = m16.astype(jnp.int32)
                c16 = plsc.cumsum(mi16)
                d16 = pos + c16 - 1
                cur = inv_v[pl.ds(g * 16, 16)]
                inv_v[pl.ds(g * 16, 16)] = cur + mi16 * (d16 - cur)
                pos = pos + jnp.sum(mi16)
            return carry
        lax.fori_loop(0, NB, pbody, 0)
        pltpu.sync_copy(inv_v, inv_hbm.at[pl.ds(base, _CHUNK)])

    return sortk(ids_flat, hists)


def _sc_scatter_zv(zz, v2, inv_flat):
    @functools.partial(
        pl.kernel,
        out_type=[
            jax.ShapeDtypeStruct((B * N, D), jnp.float32),
            jax.ShapeDtypeStruct((B * N, KVH * DH), jnp.float32),
        ],
        mesh=_sc_mesh(),
        compiler_params=pltpu.CompilerParams(needs_layout_passes=False),
        scratch_types=[
            pltpu.VMEM((_RCH,), jnp.int32),
            pltpu.VMEM((_RCH,), jnp.int32),
            pltpu.VMEM((_RCH, D), jnp.float32),
            pltpu.VMEM((_RCH, D), jnp.float32),
            pltpu.VMEM((_RCH, KVH * DH), jnp.float32),
            pltpu.VMEM((_RCH, KVH * DH), jnp.float32),
            pltpu.SemaphoreType.DMA,
        ],
    )
    def scatk(z_hbm, v_hbm, inv_hbm, zs_hbm, vs_hbm,
              idx0, idx1, bufz0, bufz1, bufv0, bufv1, sem):
        c = lax.axis_index("c")
        s = lax.axis_index("s")
        base = c * N + s * _CHUNK
        nj = _CHUNK // _RCH
        idxs = (idx0, idx1)
        for src, dst, bufs in ((z_hbm, zs_hbm, (bufz0, bufz1)),
                               (v_hbm, vs_hbm, (bufv0, bufv1))):
            pltpu.sync_copy(src.at[pl.ds(base, _RCH)], bufs[0])
            pltpu.sync_copy(inv_hbm.at[pl.ds(base, _RCH)], idxs[0])
            for j in range(nj):
                cur = j % 2
                h = pltpu.async_copy(bufs[cur], dst.at[idxs[cur]], sem)
                if j < nj - 1:
                    b0 = base + (j + 1) * _RCH
                    pltpu.sync_copy(src.at[pl.ds(b0, _RCH)], bufs[1 - cur])
                    pltpu.sync_copy(inv_hbm.at[pl.ds(b0, _RCH)],
                                    idxs[1 - cur])
                h.wait()

    return scatk(zz, v2, inv_flat)


def _sc_unsort(ys, inv_flat):
    @functools.partial(
        pl.kernel,
        out_type=jax.ShapeDtypeStruct((B * N, D), jnp.float32),
        mesh=_sc_mesh(),
        compiler_params=pltpu.CompilerParams(needs_layout_passes=False),
        scratch_types=[
            pltpu.VMEM((_RCH,), jnp.int32),
            pltpu.VMEM((_RCH,), jnp.int32),
            pltpu.VMEM((_RCH, D), jnp.float32),
            pltpu.VMEM((_RCH, D), jnp.float32),
            pltpu.SemaphoreType.DMA,
            pltpu.SemaphoreType.DMA,
            pltpu.SemaphoreType.DMA,
        ],
    )
    def unsk(ys_hbm, inv_hbm, y_hbm, idx0, idx1, buf0, buf1,
             gsem, st0, st1):
        c = lax.axis_index("c")
        s = lax.axis_index("s")
        base = c * N + s * _CHUNK
        nj = _CHUNK // _RCH
        idxs = (idx0, idx1)
        bufs = (buf0, buf1)
        stsems = (st0, st1)
        sth = [None, None]
        pltpu.sync_copy(inv_hbm.at[pl.ds(base, _RCH)], idxs[0])
        for j in range(nj):
            cur = j % 2
            if sth[cur] is not None:
                sth[cur].wait()
            gh = pltpu.async_copy(ys_hbm.at[idxs[cur]], bufs[cur], gsem)
            if j < nj - 1:
                pltpu.sync_copy(inv_hbm.at[pl.ds(base + (j + 1) * _RCH,
                                                 _RCH)], idxs[1 - cur])
            gh.wait()
            sth[cur] = pltpu.async_copy(
                bufs[cur], y_hbm.at[pl.ds(base + j * _RCH, _RCH)],
                stsems[cur])
        for h in sth:
            if h is not None:
                h.wait()

    return unsk(ys, inv_flat)


def kernel(x, W_inp, conv_w, conv_b, W_out, Wq, Wk, Wv, Wo, rms_w, W_base,
           rot, salts):
    xs = jax.lax.stop_gradient(x)
    base = xs @ W_base.T
    sims = jnp.stack([base @ rot[i] for i in range(NH)], axis=2)
    bits = sims >= 0
    masked = jnp.where(bits, salts[None, None], jnp.zeros((), dtype=jnp.int32))
    per_hash = masked[..., 0]
    for j in range(1, HD):
        per_hash = jnp.bitwise_xor(per_hash, masked[..., j])
    code = per_hash[..., 0]
    for i in range(1, NH):
        code = jnp.bitwise_xor(code, per_hash[..., i])
    bucket_ids = jnp.remainder(code, NB).astype(jnp.int32)
    inv_flat = _sc_sort(bucket_ids.reshape(B * N))

    x2 = x.reshape(B * N, D)
    T = N // TB
    zrow = jnp.zeros((B, 1, D), jnp.float32)
    xprev = jnp.concatenate([zrow, x[:, TB - 1::TB][:, :T - 1]], axis=1)
    xnext = jnp.concatenate([x[:, TB::TB], zrow], axis=1)
    xb = jnp.stack([xprev, xnext], axis=2).reshape(B * T, 2, D)

    zz, v2 = _enc_a(x2, xb, W_inp, conv_w.T, conv_b.reshape(1, D), Wv)

    zz_s, v_s = _sc_scatter_zv(zz, v2, inv_flat)
    q_s, k_s = _enc_bc(zz_s, W_out, Wq, Wk)
    o_s = _attention(q_s.reshape(B * NB, BS, H * DH),
                     k_s.reshape(B * NB, BS, KVH * DH),
                     v_s.reshape(B * NB, BS, KVH * DH)).reshape(B * N, H * DH)

    y_s = _outproj(o_s, Wo, rms_w.reshape(1, D))
    return _sc_unsort(y_s, inv_flat).reshape(B, N, D)


# speedup vs baseline: 1.0066x; 1.0066x over previous
import functools

import jax
import jax.numpy as jnp
from jax import lax
from jax.experimental import pallas as pl
from jax.experimental.pallas import tpu as pltpu
from jax.experimental.pallas import tpu_sc as plsc

B, N, D = 2, 4096, 2048
H, KVH = 16, 4
DH = D // H
HD, NH = 8, 4
BS = 128
NB = (N + BS - 1) // BS
REP = H // KVH

TB = 256
TT = (B * N) // TB
TBO = 512
TTO = (B * N) // TBO


def _silu(t):
    return t * jax.nn.sigmoid(t)


def _full(shape):
    return pl.BlockSpec(shape, lambda i: (0,) * len(shape))


def _encA_kernel(x_ref, xb_ref, winp_ref, cw_ref, cb_ref, wv_ref,
                 zz_ref, v_ref):
    x = x_ref[...]
    xb = xb_ref[0]
    zg = lax.dot_general(x, winp_ref[...], (((1,), (1,)), ((), ())))
    z = zg[:, :D]
    gate = zg[:, D:]
    zb = lax.dot_general(xb, winp_ref[:D], (((1,), (1,)), ((), ())))
    z_up = jnp.concatenate([zb[0:1], z[:-1]], axis=0)
    z_dn = jnp.concatenate([z[1:], zb[1:2]], axis=0)
    zc = (z_up * cw_ref[0:1] + z * cw_ref[1:2] + z_dn * cw_ref[2:3]
          + cb_ref[0:1])
    zz_ref[...] = _silu(zc) * _silu(gate)
    v_ref[...] = lax.dot_general(x, wv_ref[...], (((1,), (1,)), ((), ())))


def _encB_kernel(zz_ref, wout_ref, qkg_ref):
    qkg_ref[...] = lax.dot_general(zz_ref[...], wout_ref[...],
                                   (((1,), (1,)), ((), ())))


def _encC_kernel(qg_ref, kg_ref, wq_ref, wk_ref, q_ref, k_ref):
    q_ref[...] = lax.dot_general(qg_ref[...], wq_ref[...],
                                 (((1,), (1,)), ((), ())))
    k_ref[...] = lax.dot_general(kg_ref[...], wk_ref[...],
                                 (((1,), (1,)), ((), ())))


def _enc_a(x2, xb, winp_t, cw, cb2, wv_t):
    return pl.pallas_call(
        _encA_kernel,
        grid=(TT,),
        in_specs=[
            pl.BlockSpec((TB, D), lambda i: (i, 0)),
            pl.BlockSpec((1, 2, D), lambda i: (i, 0, 0)),
            _full((2 * D, D)),
            _full((3, D)),
            _full((1, D)),
            _full((KVH * DH, D)),
        ],
        out_specs=[
            pl.BlockSpec((TB, D), lambda i: (i, 0)),
            pl.BlockSpec((TB, KVH * DH), lambda i: (i, 0)),
        ],
        out_shape=[
            jax.ShapeDtypeStruct((B * N, D), jnp.float32),
            jax.ShapeDtypeStruct((B * N, KVH * DH), jnp.float32),
        ],
    )(x2, xb, winp_t, cw, cb2, wv_t)


def _enc_bc(zz, wout_t, wq_t, wk_t):
    qkg = pl.pallas_call(
        _encB_kernel,
        grid=(TT,),
        in_specs=[
            pl.BlockSpec((TB, D), lambda i: (i, 0)),
            _full((2 * D, D)),
        ],
        out_specs=pl.BlockSpec((TB, 2 * D), lambda i: (i, 0)),
        out_shape=jax.ShapeDtypeStruct((B * N, 2 * D), jnp.float32),
    )(zz, wout_t)
    q2, k2 = pl.pallas_call(
        _encC_kernel,
        grid=(TT,),
        in_specs=[
            pl.BlockSpec((TB, D), lambda i: (i, 0)),
            pl.BlockSpec((TB, D), lambda i: (i, 1)),
            _full((H * DH, D)),
            _full((KVH * DH, D)),
        ],
        out_specs=[
            pl.BlockSpec((TB, H * DH), lambda i: (i, 0)),
            pl.BlockSpec((TB, KVH * DH), lambda i: (i, 0)),
        ],
        out_shape=[
            jax.ShapeDtypeStruct((B * N, H * DH), jnp.float32),
            jax.ShapeDtypeStruct((B * N, KVH * DH), jnp.float32),
        ],
    )(qkg, qkg, wq_t, wk_t)
    return q2, k2


def _att_kernel(q_ref, k_ref, v_ref, o_ref):
    q = q_ref[0]
    k = k_ref[0]
    v = v_ref[0]
    scale = 1.0 / jnp.sqrt(jnp.float32(DH))
    for g in range(KVH):
        kh = k[:, g * DH:(g + 1) * DH]
        vh = v[:, g * DH:(g + 1) * DH]
        q4 = jnp.concatenate(
            [q[:, (g * REP + j) * DH:(g * REP + j + 1) * DH]
             for j in range(REP)], axis=0)
        s = jax.lax.dot_general(q4, kh, (((1,), (1,)), ((), ()))) * scale
        m = jnp.max(s, axis=1, keepdims=True)
        p = jnp.exp(s - m)
        a = p / jnp.sum(p, axis=1, keepdims=True)
        o4 = jnp.dot(a, vh)
        for j in range(REP):
            h = g * REP + j
            o_ref[0, :, h * DH:(h + 1) * DH] = o4[j * BS:(j + 1) * BS]


def _attention(qs, ks, vs):
    return pl.pallas_call(
        _att_kernel,
        grid=(B * NB,),
        in_specs=[
            pl.BlockSpec((1, BS, H * DH), lambda i: (i, 0, 0)),
            pl.BlockSpec((1, BS, KVH * DH), lambda i: (i, 0, 0)),
            pl.BlockSpec((1, BS, KVH * DH), lambda i: (i, 0, 0)),
        ],
        out_specs=pl.BlockSpec((1, BS, H * DH), lambda i: (i, 0, 0)),
        out_shape=jax.ShapeDtypeStruct((B * NB, BS, H * DH), jnp.float32),
    )(qs, ks, vs)


def _out_kernel(o_ref, wo_ref, rw_ref, y_ref):
    oo = lax.dot_general(o_ref[...], wo_ref[...], (((1,), (1,)), ((), ())))
    rms = jnp.sqrt(jnp.mean(oo * oo, axis=-1, keepdims=True) + 1e-8)
    y_ref[...] = oo / rms * rw_ref[0:1]


def _outproj(o_s, wo_t, rw2):
    return pl.pallas_call(
        _out_kernel,
        grid=(TTO,),
        in_specs=[
            pl.BlockSpec((TBO, D), lambda i: (i, 0)),
            pl.BlockSpec((D, D), lambda i: (0, 0)),
            pl.BlockSpec((1, D), lambda i: (0, 0)),
        ],
        out_specs=pl.BlockSpec((TBO, D), lambda i: (i, 0)),
        out_shape=jax.ShapeDtypeStruct((B * N, D), jnp.float32),
    )(o_s, wo_t, rw2)


_CHUNK = (B * N) // 32
_RCH = 32


def _sc_mesh():
    return plsc.VectorSubcoreMesh(core_axis_name="c", subcore_axis_name="s")


def _sc_sort(ids_flat):
    nv = _CHUNK // 16

    @functools.partial(
        pl.kernel,
        out_type=jax.ShapeDtypeStruct((32 * NB,), jnp.int32),
        mesh=_sc_mesh(),
        compiler_params=pltpu.CompilerParams(needs_layout_passes=False),
        scratch_types=[
            pltpu.VMEM((_CHUNK,), jnp.int32),
            pltpu.VMEM((NB,), jnp.int32),
        ],
    )
    def histk(ids_hbm, hists_hbm, ids_v, hist_v):
        c = lax.axis_index("c")
        s = lax.axis_index("s")
        w = c * 16 + s
        base = c * N + s * _CHUNK
        pltpu.sync_copy(ids_hbm.at[pl.ds(base, _CHUNK)], ids_v)
        lanes = lax.iota(jnp.int32, 16)

        def hbody(b, carry):
            cnt = jnp.int32(0)
            for g in range(nv):
                cnt += jnp.sum((ids_v[pl.ds(g * 16, 16)] == b)
                               .astype(jnp.int32))
            half = b // 16
            lane = b - half * 16
            m_lane = (lanes == lane).astype(jnp.int32)
            for hh in range(2):
                eq = jnp.int32(1) - jnp.minimum(jnp.abs(half - hh), 1)
                cur = hist_v[pl.ds(hh * 16, 16)]
                hist_v[pl.ds(hh * 16, 16)] = cur + m_lane * eq * (cnt - cur)
            return carry
        lax.fori_loop(0, NB, hbody, 0)
        pltpu.sync_copy(hist_v, hists_hbm.at[pl.ds(w * NB, NB)])

    hists = histk(ids_flat)

    @functools.partial(
        pl.kernel,
        out_type=jax.ShapeDtypeStruct((B * N,), jnp.int32),
        mesh=_sc_mesh(),
        compiler_params=pltpu.CompilerParams(needs_layout_passes=False),
        scratch_types=[
            pltpu.VMEM((_CHUNK,), jnp.int32),
            pltpu.VMEM((16 * NB,), jnp.int32),
            pltpu.VMEM((_CHUNK,), jnp.int32),
        ],
    )
    def sortk(ids_hbm, hists_hbm, inv_hbm, ids_v, allh_v, inv_v):
        c = lax.axis_index("c")
        s = lax.axis_index("s")
        base = c * N + s * _CHUNK
        pltpu.sync_copy(ids_hbm.at[pl.ds(base, _CHUNK)], ids_v)
        pltpu.sync_copy(hists_hbm.at[pl.ds(c * 16 * NB, 16 * NB)], allh_v)
        lanes = lax.iota(jnp.int32, 16)

        tot = [jnp.zeros((16,), jnp.int32) for _ in range(2)]
        part = [jnp.zeros((16,), jnp.int32) for _ in range(2)]
        for t in range(16):
            sel = jnp.minimum(jnp.maximum(s - t, 0), 1)
            for hh in range(2):
                row = allh_v[pl.ds(t * NB + hh * 16, 16)]
                tot[hh] = tot[hh] + row
                part[hh] = part[hh] + row * sel
        excl = []
        run = c * N
        for hh in range(2):
            cs = plsc.cumsum(tot[hh])
            excl.append(run + cs - tot[hh] + part[hh])
            run = run + jnp.sum(tot[hh])

        def pbody(b, carry):
            half = b // 16
            lane = b - half * 16
            m_lane = (lanes == lane).astype(jnp.int32)
            offv = excl[0] + (excl[1] - excl[0]) * half
            pos = jnp.sum(offv * m_lane)
            for g in range(nv):
                ids16 = ids_v[pl.ds(g * 16, 16)]
                m16 = (ids16 == b)
                mi16 = m16.astype(jnp.int32)
                c16 = plsc.cumsum(mi16)
                d16 = pos + c16 - 1
                cur = inv_v[pl.ds(g * 16, 16)]
                inv_v[pl.ds(g * 16, 16)] = cur + mi16 * (d16 - cur)
                pos = pos + jnp.sum(mi16)
            return carry
        lax.fori_loop(0, NB, pbody, 0)
        pltpu.sync_copy(inv_v, inv_hbm.at[pl.ds(base, _CHUNK)])

    return sortk(ids_flat, hists)


def _sc_scatter_zv(zz, v2, inv_flat):
    @functools.partial(
        pl.kernel,
        out_type=[
            jax.ShapeDtypeStruct((B * N, D), jnp.float32),
            jax.ShapeDtypeStruct((B * N, KVH * DH), jnp.float32),
        ],
        mesh=_sc_mesh(),
        compiler_params=pltpu.CompilerParams(needs_layout_passes=False),
        scratch_types=[
            pltpu.VMEM((_RCH,), jnp.int32),
            pltpu.VMEM((_RCH, D), jnp.float32),
            pltpu.VMEM((_RCH, KVH * DH), jnp.float32),
            pltpu.SemaphoreType.DMA,
        ],
    )
    def scatk(z_hbm, v_hbm, inv_hbm, zs_hbm, vs_hbm, idx_v, bufz, bufv, sem):
        c = lax.axis_index("c")
        s = lax.axis_index("s")
        base = c * N + s * _CHUNK
        for src, dst, buf in ((z_hbm, zs_hbm, bufz),
                              (v_hbm, vs_hbm, bufv)):
            for j in range(_CHUNK // _RCH):
                b0 = base + j * _RCH
                pltpu.sync_copy(src.at[pl.ds(b0, _RCH)], buf)
                pltpu.sync_copy(inv_hbm.at[pl.ds(b0, _RCH)], idx_v)
                pltpu.async_copy(buf, dst.at[idx_v], sem).wait()

    return scatk(zz, v2, inv_flat)


def _sc_unsort(ys, inv_flat):
    @functools.partial(
        pl.kernel,
        out_type=jax.ShapeDtypeStruct((B * N, D), jnp.float32),
        mesh=_sc_mesh(),
        compiler_params=pltpu.CompilerParams(needs_layout_passes=False),
        scratch_types=[
            pltpu.VMEM((_RCH,), jnp.int32),
            pltpu.VMEM((_RCH, D), jnp.float32),
            pltpu.SemaphoreType.DMA,
        ],
    )
    def unsk(ys_hbm, inv_hbm, y_hbm, idx_v, buf, sem):
        c = lax.axis_index("c")
        s = lax.axis_index("s")
        base = c * N + s * _CHUNK

        for j in range(_CHUNK // _RCH):
            b0 = base + j * _RCH
            pltpu.sync_copy(inv_hbm.at[pl.ds(b0, _RCH)], idx_v)
            pltpu.async_copy(ys_hbm.at[idx_v], buf, sem).wait()
            pltpu.sync_copy(buf, y_hbm.at[pl.ds(b0, _RCH)])

    return unsk(ys, inv_flat)


def kernel(x, W_inp, conv_w, conv_b, W_out, Wq, Wk, Wv, Wo, rms_w, W_base,
           rot, salts):
    xs = jax.lax.stop_gradient(x)
    base = xs @ W_base.T
    sims = jnp.stack([base @ rot[i] for i in range(NH)], axis=2)
    bits = sims >= 0
    masked = jnp.where(bits, salts[None, None], jnp.zeros((), dtype=jnp.int32))
    per_hash = masked[..., 0]
    for j in range(1, HD):
        per_hash = jnp.bitwise_xor(per_hash, masked[..., j])
    code = per_hash[..., 0]
    for i in range(1, NH):
        code = jnp.bitwise_xor(code, per_hash[..., i])
    bucket_ids = jnp.remainder(code, NB).astype(jnp.int32)
    inv_flat = _sc_sort(bucket_ids.reshape(B * N))

    x2 = x.reshape(B * N, D)
    T = N // TB
    zrow = jnp.zeros((B, 1, D), jnp.float32)
    xprev = jnp.concatenate([zrow, x[:, TB - 1::TB][:, :T - 1]], axis=1)
    xnext = jnp.concatenate([x[:, TB::TB], zrow], axis=1)
    xb = jnp.stack([xprev, xnext], axis=2).reshape(B * T, 2, D)

    zz, v2 = _enc_a(x2, xb, W_inp, conv_w.T, conv_b.reshape(1, D), Wv)

    zz_s, v_s = _sc_scatter_zv(zz, v2, inv_flat)
    q_s, k_s = _enc_bc(zz_s, W_out, Wq, Wk)
    o_s = _attention(q_s.reshape(B * NB, BS, H * DH),
                     k_s.reshape(B * NB, BS, KVH * DH),
                     v_s.reshape(B * NB, BS, KVH * DH)).reshape(B * N, H * DH)

    y_s = _outproj(o_s, Wo, rms_w.reshape(1, D))
    return _sc_unsort(y_s, inv_flat).reshape(B, N, D)
